# baseline (speedup 1.0000x reference)
import functools

import jax
import jax.numpy as jnp
from jax import lax
from jax.experimental import pallas as pl
from jax.experimental.pallas import tpu as pltpu
from jax.experimental.pallas import tpu_sc as plsc

_VOCAB = 100000
_D = 64
_NT = 5
_SEQ = 200
_NG = _SEQ - _NT
_IDXP = 208
_L = 16
_DEPTH = 4


def _build(B, NC, NS):
    NW = NC * NS
    bpw = B // NW
    T = bpw // _DEPTH
    mesh = plsc.VectorSubcoreMesh(core_axis_name="c", subcore_axis_name="s")

    @functools.partial(
        pl.kernel,
        out_type=jax.ShapeDtypeStruct((B * _SEQ, _D), jnp.float32),
        mesh=mesh,
        compiler_params=pltpu.CompilerParams(use_tc_tiling_on_sc=False),
        scratch_types=[
            pltpu.VMEM((bpw, _IDXP), jnp.int32),
            pltpu.VMEM((bpw, _NT * _D), jnp.float32),
            [pltpu.VMEM((_SEQ, _D), jnp.float32)] * _DEPTH,
            pltpu.VMEM((_NT * _D,), jnp.float32),
            pltpu.VMEM((_NT * _D,), jnp.float32),
            [pltpu.SemaphoreType.DMA] * _DEPTH,
            [pltpu.SemaphoreType.DMA] * _DEPTH,
        ],
    )
    def k(idx_hbm, table_hbm, samp_hbm, var_hbm, avg_hbm, out_hbm,
          idxs_v, samps_v, bufs, var_v, avg_v, sgs, sws):
        wid = lax.axis_index("s") * NC + lax.axis_index("c")
        b0 = wid * bpw
        pltpu.sync_copy(var_hbm, var_v)
        pltpu.sync_copy(avg_hbm, avg_v)
        pltpu.sync_copy(idx_hbm.at[pl.ds(b0, bpw)], idxs_v)
        pltpu.sync_copy(samp_hbm.at[pl.ds(b0, bpw)], samps_v)

        def gather_issue(g, buf, sem):
            pltpu.async_copy(table_hbm.at[idxs_v.at[g, pl.ds(0, 128)]],
                             buf.at[pl.ds(0, 128)], sem)
            pltpu.async_copy(table_hbm.at[idxs_v.at[g, pl.ds(128, 72)]],
                             buf.at[pl.ds(128, 72)], sem)

        def gather_wait(buf, sem):
            pltpu.make_async_copy(table_hbm.at[pl.ds(0, 128)],
                                  buf.at[pl.ds(0, 128)], sem).wait()
            pltpu.make_async_copy(table_hbm.at[pl.ds(0, 72)],
                                  buf.at[pl.ds(128, 72)], sem).wait()

        def write_issue(g, buf, sem):
            pltpu.async_copy(buf, out_hbm.at[pl.ds((b0 + g) * _SEQ, _SEQ)],
                             sem)

        def write_wait(buf, sem):
            pltpu.make_async_copy(buf, out_hbm.at[pl.ds(0, _SEQ)], sem).wait()

        def prefix(g, buf):
            for j in range(_NT * _D // _L):
                r, c = divmod(j, _D // _L)
                sl = pl.ds(j * _L, _L)
                buf[_NG + r, pl.ds(c * _L, _L)] = (
                    samps_v[g, sl] * var_v[sl] + avg_v[sl])

        for s in range(_DEPTH - 1):
            gather_issue(s, bufs[s], sgs[s])

        def body(t, carry):
            for s in range(_DEPTH):
                g = _DEPTH * t + s
                sprev = (s - 1) % _DEPTH
                gather_wait(bufs[s], sgs[s])
                prefix(g, bufs[s])
                write_issue(g, bufs[s], sws[s])
                if s == 0:
                    @pl.when(t > 0)
                    def _():
                        write_wait(bufs[sprev], sws[sprev])

                    gather_issue(g + _DEPTH - 1, bufs[sprev], sgs[sprev])
                else:
                    @pl.when(t < T - 1)
                    def _():
                        write_wait(bufs[sprev], sws[sprev])
                        gather_issue(g + _DEPTH - 1, bufs[sprev], sgs[sprev])
            return carry

        lax.fori_loop(0, T, body, 0)
        for s in range(_DEPTH):
            write_wait(bufs[s], sws[s])

    return k


def kernel(tokens, table, avg, var):
    B = tokens.shape[0]
    idx = jnp.pad(tokens[:, _NT:], ((0, 0), (0, _IDXP - _NG)))
    sample = jax.random.normal(jax.random.key(1), (B, _NT, _D),
                               dtype=jnp.float32)
    info = plsc.get_sparse_core_info()
    k = _build(B, info.num_cores, info.num_subcores)
    out = k(idx, table, sample.reshape(B, _NT * _D),
            var.reshape(_NT * _D), avg.reshape(_NT * _D))
    return out.reshape(B, _SEQ, _D)


# speedup vs baseline: 1.6326x; 1.6326x over previous
import functools

import jax
import jax.numpy as jnp
from jax import lax
from jax.experimental import pallas as pl
from jax.experimental.pallas import tpu as pltpu
from jax.experimental.pallas import tpu_sc as plsc

_VOCAB = 100000
_D = 64
_NT = 5
_SEQ = 200
_NG = _SEQ - _NT
_L = 16


def _build(B, NC, NS):
    NW = NC * NS
    bpw = B // NW
    mesh = plsc.VectorSubcoreMesh(core_axis_name="c", subcore_axis_name="s")

    @functools.partial(
        pl.kernel,
        out_type=jax.ShapeDtypeStruct((B * _SEQ, _D), jnp.float32),
        mesh=mesh,
        compiler_params=pltpu.CompilerParams(use_tc_tiling_on_sc=False),
        scratch_types=[
            pltpu.VMEM((bpw, _SEQ), jnp.int32),
            pltpu.VMEM((bpw, _NT * _D), jnp.float32),
            pltpu.VMEM((_SEQ + _NT, _D), jnp.float32),
            pltpu.VMEM((_NT * _D,), jnp.float32),
            pltpu.VMEM((_NT * _D,), jnp.float32),
            pltpu.SemaphoreType.DMA,
        ],
    )
    def k(tok_hbm, table_hbm, samp_hbm, var_hbm, avg_hbm, out_hbm,
          idxs_v, samps_v, buf, var_v, avg_v, sg):
        wid = lax.axis_index("s") * NC + lax.axis_index("c")
        b0 = wid * bpw
        pltpu.sync_copy(var_hbm, var_v)
        pltpu.sync_copy(avg_hbm, avg_v)
        pltpu.sync_copy(tok_hbm.at[pl.ds(b0, bpw)], idxs_v)
        pltpu.sync_copy(samp_hbm.at[pl.ds(b0, bpw)], samps_v)

        def body(g, carry):
            c1 = pltpu.async_copy(table_hbm.at[idxs_v.at[g, pl.ds(0, 128)]],
                                  buf.at[pl.ds(0, 128)], sg)
            c2 = pltpu.async_copy(table_hbm.at[idxs_v.at[g, pl.ds(128, 72)]],
                                  buf.at[pl.ds(128, 72)], sg)
            for j in range(_NT * _D // _L):
                r, c = divmod(j, _D // _L)
                sl = pl.ds(j * _L, _L)
                buf[_SEQ + r, pl.ds(c * _L, _L)] = (
                    samps_v[g, sl] * var_v[sl] + avg_v[sl])
            c1.wait()
            c2.wait()
            pltpu.sync_copy(buf.at[pl.ds(_NT, _SEQ)],
                            out_hbm.at[pl.ds((b0 + g) * _SEQ, _SEQ)])
            return carry

        lax.fori_loop(0, bpw, body, 0)

    return k


def kernel(tokens, table, avg, var):
    B = tokens.shape[0]
    sample = jax.random.normal(jax.random.key(1), (B, _NT, _D),
                               dtype=jnp.float32)
    info = plsc.get_sparse_core_info()
    k = _build(B, info.num_cores, info.num_subcores)
    out = k(tokens, table, sample.reshape(B, _NT * _D),
            var.reshape(_NT * _D), avg.reshape(_NT * _D))
    return out.reshape(B, _SEQ, _D)
